# baseline (speedup 1.0000x reference)
import functools

import numpy as np
import jax
import jax.numpy as jnp
from jax import lax
from jax.experimental import pallas as pl
from jax.experimental.pallas import tpu as pltpu, tpu_sc as plsc

B = 16384
D_CUST, D_CITY, D_REV, D_CC, D_TS, D_CYC, D_RT = 64, 16, 32, 16, 32, 8, 12
D_RT_G = 16
D_OUT = D_CUST + D_CITY + D_REV + 1 + D_CC + D_TS + D_CYC + D_RT
C_CITY = D_CUST
C_REV = C_CITY + D_CITY
C_NORM = C_REV + D_REV
C_CC = C_NORM + 1
C_TS = C_CC + D_CC
C_CYC = C_TS + D_TS
C_RT = C_CYC + D_CYC
NBOUND = 100

_PI = float(np.pi)


def _bsearch(bounds_ref, v):
    lo = jnp.zeros((16,), jnp.int32)
    w = 64
    while w >= 1:
        cand = lo + w
        g = plsc.load_gather(bounds_ref, [jnp.minimum(cand - 1, NBOUND - 1)])
        ok = (cand <= NBOUND) & (g < v)
        lo = jnp.where(ok, cand, lo)
        w //= 2
    return lo


def _sincos_2pi(f):
    q = f * 4.0
    k = (q + 0.5).astype(jnp.int32)
    t = q - k.astype(jnp.float32)
    th = t * jnp.float32(_PI / 2)
    th2 = th * th
    s = th * (1.0 + th2 * (-1.0 / 6.0 + th2 * (1.0 / 120.0 + th2 * (-1.0 / 5040.0))))
    c = 1.0 + th2 * (-0.5 + th2 * (1.0 / 24.0 + th2 * (-1.0 / 720.0)))
    k2 = k & 3
    sin_o = jnp.where(k2 == 0, s, jnp.where(k2 == 1, c, jnp.where(k2 == 2, -s, -c)))
    cos_o = jnp.where(k2 == 0, c, jnp.where(k2 == 1, -s, jnp.where(k2 == 2, -c, s)))
    return sin_o, cos_o


def _make_sc_kernel():
    info = plsc.get_sparse_core_info()
    NC, NS = info.num_cores, info.num_subcores
    NW = NC * NS
    R = B // NW
    RH = R // 2
    mesh = plsc.VectorSubcoreMesh(core_axis_name="c", subcore_axis_name="s")

    @functools.partial(
        pl.kernel,
        mesh=mesh,
        out_type=jax.ShapeDtypeStruct((B, D_OUT), jnp.float32),
        compiler_params=pltpu.CompilerParams(use_tc_tiling_on_sc=False,
                                             needs_layout_passes=False),
        scratch_types=[
            pltpu.VMEM((RH,), jnp.int32),
            pltpu.VMEM((RH,), jnp.int32),
            pltpu.VMEM((RH,), jnp.float32),
            pltpu.VMEM((RH,), jnp.int32),
            pltpu.VMEM((NBOUND,), jnp.float32),
            pltpu.VMEM((NBOUND,), jnp.float32),
            pltpu.VMEM((64,), jnp.float32),
            pltpu.VMEM((RH,), jnp.int32),
            pltpu.VMEM((RH,), jnp.int32),
            pltpu.VMEM((RH,), jnp.int32),
            pltpu.VMEM((RH,), jnp.int32),
            pltpu.VMEM((RH,), jnp.int32),
            pltpu.VMEM((RH, D_OUT), jnp.float32),
            pltpu.VMEM((RH, 2 * D_CUST), jnp.float32),
            pltpu.VMEM((RH, D_CITY), jnp.float32),
            pltpu.VMEM((RH, D_REV), jnp.float32),
            pltpu.VMEM((RH, D_CC), jnp.float32),
            pltpu.VMEM((RH, D_TS), jnp.float32),
            pltpu.VMEM((RH, D_RT_G), jnp.float32),
            pltpu.SemaphoreType.DMA,
            pltpu.SemaphoreType.DMA,
            pltpu.SemaphoreType.DMA,
            pltpu.SemaphoreType.DMA,
            pltpu.SemaphoreType.DMA,
            pltpu.SemaphoreType.DMA,
            pltpu.SemaphoreType.DMA,
        ],
    )
    def sc_kernel(cid_hbm, city_hbm, rev_hbm, ts_hbm, cust_t, city_t, rev_t,
                  cc_t, rt_t, ts_t, revb_hbm, tsb_hbm, consts_hbm, out_hbm,
                  cid_v, city_v, rev_v, ts_v, revb_v, tsb_v, consts_v,
                  revbk_v, cc_i, tsbk_v, rt_i, cidh_v,
                  comb, cust_rows, city_rows, rev_rows, cc_rows, ts_rows, rt_rows,
                  sem_cu, sem_ci, sem_re, sem_cc, sem_ts, sem_rt, sem_o):
        wid = lax.axis_index("s") * NC + lax.axis_index("c")

        pltpu.sync_copy(revb_hbm, revb_v)
        pltpu.sync_copy(tsb_hbm, tsb_v)
        pltpu.sync_copy(consts_hbm, consts_v)
        rev_mean = consts_v[pl.ds(0, 16)]
        rev_std = consts_v[pl.ds(16, 16)]
        ts_mean = consts_v[pl.ds(32, 16)]
        ts_std = consts_v[pl.ds(48, 16)]

        out_h = None
        for p in range(2):
            base = wid * R + p * RH
            pltpu.sync_copy(cid_hbm.at[pl.ds(base, RH)], cid_v)
            pltpu.sync_copy(city_hbm.at[pl.ds(base, RH)], city_v)
            pltpu.sync_copy(rev_hbm.at[pl.ds(base, RH)], rev_v)
            pltpu.sync_copy(ts_hbm.at[pl.ds(base, RH)], ts_v)

            if out_h is not None:
                out_h.wait()

            def halve(i, _):
                o = i * 16
                cidh_v[pl.ds(o, 16)] = lax.shift_right_logical(
                    cid_v[pl.ds(o, 16)], 1)
                return _

            lax.fori_loop(0, RH // 16, halve, None)
            g_cust = pltpu.async_copy(cust_t.at[cidh_v], cust_rows, sem_cu)
            g_city = pltpu.async_copy(city_t.at[city_v], city_rows, sem_ci)

            def chunk(i, _):
                o = i * 16
                rows = o + lax.iota(jnp.int32, 16)
                cid = cid_v[pl.ds(o, 16)]
                city = city_v[pl.ds(o, 16)]
                rev = rev_v[pl.ds(o, 16)]
                ts = ts_v[pl.ds(o, 16)]

                rbk = _bsearch(revb_v, rev)
                revbk_v[pl.ds(o, 16)] = rbk
                plsc.store_scatter(comb, [rows, jnp.full((16,), C_NORM, jnp.int32)],
                                   (rev - rev_mean) / rev_std)

                cc = ((cid.astype(jnp.uint32) * jnp.uint32(2654435761)) ^
                      (city.astype(jnp.uint32) * jnp.uint32(40503))) % jnp.uint32(5000)
                cc_i[pl.ds(o, 16)] = cc.astype(jnp.int32)

                ts_f = ts.astype(jnp.float32)
                tbk = _bsearch(tsb_v, (ts_f - ts_mean) / ts_std)
                tsbk_v[pl.ds(o, 16)] = tbk
                ts_hour = ts.astype(jnp.uint32) // jnp.uint32(3600)
                rt = ((rbk.astype(jnp.uint32) * jnp.uint32(2654435761)) ^
                      (ts_hour * jnp.uint32(40503))) % jnp.uint32(3000)
                rt_i[pl.ds(o, 16)] = rt.astype(jnp.int32)

                days = ts_f / 86400.0
                dow = jnp.mod(days + 3.0, 7.0) / 7.0
                woy = jnp.mod(days / 7.0, 52.0) / 52.0
                dom = jnp.mod(days, 30.44) / 30.44
                moy = jnp.mod(days / 30.44, 12.0) / 12.0
                col = C_CYC
                for fr in (dow, woy, dom, moy):
                    s, c = _sincos_2pi(fr)
                    plsc.store_scatter(comb, [rows, jnp.full((16,), col, jnp.int32)], s)
                    plsc.store_scatter(comb, [rows, jnp.full((16,), col + 1, jnp.int32)], c)
                    col += 2
                return _

            lax.fori_loop(0, RH // 16, chunk, None)

            g_rev = pltpu.async_copy(rev_t.at[revbk_v], rev_rows, sem_re)
            g_cc = pltpu.async_copy(cc_t.at[cc_i], cc_rows, sem_cc)
            g_ts = pltpu.async_copy(ts_t.at[tsbk_v], ts_rows, sem_ts)
            g_rt = pltpu.async_copy(rt_t.at[rt_i], rt_rows, sem_rt)
            for g in (g_cust, g_city, g_rev, g_cc, g_ts, g_rt):
                g.wait()

            def fixup(i, _):
                o = i * 16
                rows = o + lax.iota(jnp.int32, 16)
                half = (cid_v[pl.ds(o, 16)] & 1) * D_CUST
                for c in range(D_CUST):
                    v = plsc.load_gather(cust_rows, [rows, half + c])
                    plsc.store_scatter(
                        comb, [rows, jnp.full((16,), c, jnp.int32)], v)
                for src, w, c0 in ((city_rows, D_CITY, C_CITY),
                                   (rev_rows, D_REV, C_REV),
                                   (cc_rows, D_CC, C_CC), (ts_rows, D_TS, C_TS),
                                   (rt_rows, D_RT, C_RT)):
                    for c in range(w):
                        v = plsc.load_gather(src, [rows, jnp.full((16,), c, jnp.int32)])
                        plsc.store_scatter(
                            comb, [rows, jnp.full((16,), c0 + c, jnp.int32)], v)
                return _

            lax.fori_loop(0, RH // 16, fixup, None)

            out_h = pltpu.async_copy(comb, out_hbm.at[pl.ds(base, RH)], sem_o)
        out_h.wait()

    return sc_kernel


def kernel(customer_id, city_id, revenue, timestamp, cust_table, city_table,
           rev_table, cc_table, rt_table, ts_table, rev_boundaries,
           ts_boundaries, rev_mean, rev_std, ts_mean, ts_std):
    consts = jnp.concatenate([
        jnp.full((16,), rev_mean, jnp.float32),
        jnp.full((16,), rev_std, jnp.float32),
        jnp.full((16,), ts_mean, jnp.float32),
        jnp.full((16,), ts_std, jnp.float32),
    ])
    rt_pad = jnp.pad(rt_table, ((0, 0), (0, D_RT_G - D_RT)))
    cust_pairs = jnp.pad(cust_table, ((0, 1), (0, 0))).reshape(-1, 2 * D_CUST)
    sc = _make_sc_kernel()
    return sc(customer_id.astype(jnp.int32), city_id.astype(jnp.int32),
              revenue, timestamp.astype(jnp.int32), cust_pairs, city_table,
              rev_table, cc_table, rt_pad, ts_table, rev_boundaries,
              ts_boundaries, consts)


# speedup vs baseline: 1.4748x; 1.4748x over previous
import functools

import numpy as np
import jax
import jax.numpy as jnp
from jax import lax
from jax.experimental import pallas as pl
from jax.experimental.pallas import tpu as pltpu, tpu_sc as plsc

B = 16384
D_CUST, D_CITY, D_REV, D_CC, D_TS, D_CYC, D_RT = 64, 16, 32, 16, 32, 8, 12
D_PAD = 128
D_OUT = D_CUST + D_CITY + D_REV + 1 + D_CC + D_TS + D_CYC + D_RT
C_CITY = D_CUST
C_REV = C_CITY + D_CITY
C_NORM = C_REV + D_REV
C_CC = C_NORM + 1
C_TS = C_CC + D_CC
C_CYC = C_TS + D_TS
C_RT = C_CYC + D_CYC
NBOUND = 100

_PI = float(np.pi)


def _bsearch(bounds_ref, v):
    lo = jnp.zeros((16,), jnp.int32)
    w = 64
    while w >= 1:
        cand = lo + w
        g = plsc.load_gather(bounds_ref, [jnp.minimum(cand - 1, NBOUND - 1)])
        ok = (cand <= NBOUND) & (g < v)
        lo = jnp.where(ok, cand, lo)
        w //= 2
    return lo


def _sincos_2pi(f):
    q = f * 4.0
    k = (q + 0.5).astype(jnp.int32)
    t = q - k.astype(jnp.float32)
    th = t * jnp.float32(_PI / 2)
    th2 = th * th
    s = th * (1.0 + th2 * (-1.0 / 6.0 + th2 * (1.0 / 120.0 + th2 * (-1.0 / 5040.0))))
    c = 1.0 + th2 * (-0.5 + th2 * (1.0 / 24.0 + th2 * (-1.0 / 720.0)))
    k2 = k & 3
    sin_o = jnp.where(k2 == 0, s, jnp.where(k2 == 1, c, jnp.where(k2 == 2, -s, -c)))
    cos_o = jnp.where(k2 == 0, c, jnp.where(k2 == 1, -s, jnp.where(k2 == 2, -c, s)))
    return sin_o, cos_o


def _make_sc_kernel():
    info = plsc.get_sparse_core_info()
    NC, NS = info.num_cores, info.num_subcores
    NW = NC * NS
    R = B // NW
    RH = 64
    NP = R // RH
    mesh = plsc.VectorSubcoreMesh(core_axis_name="c", subcore_axis_name="s")

    @functools.partial(
        pl.kernel,
        mesh=mesh,
        out_type=jax.ShapeDtypeStruct((B, D_OUT), jnp.float32),
        compiler_params=pltpu.CompilerParams(use_tc_tiling_on_sc=True,
                                             needs_layout_passes=False),
        scratch_types=[
            pltpu.VMEM((RH,), jnp.int32),
            pltpu.VMEM((RH,), jnp.int32),
            pltpu.VMEM((RH,), jnp.float32),
            pltpu.VMEM((RH,), jnp.int32),
            pltpu.VMEM((NBOUND,), jnp.float32),
            pltpu.VMEM((NBOUND,), jnp.float32),
            pltpu.VMEM((64,), jnp.float32),
            pltpu.VMEM((RH,), jnp.int32),
            pltpu.VMEM((RH,), jnp.int32),
            pltpu.VMEM((RH,), jnp.int32),
            pltpu.VMEM((RH,), jnp.int32),
            pltpu.VMEM((RH, D_OUT), jnp.float32),
            pltpu.VMEM((RH, D_PAD), jnp.float32),
            pltpu.VMEM((RH, D_PAD), jnp.float32),
            pltpu.VMEM((RH, D_PAD), jnp.float32),
            pltpu.VMEM((RH, D_PAD), jnp.float32),
            pltpu.VMEM((RH, D_PAD), jnp.float32),
            pltpu.VMEM((RH, D_PAD), jnp.float32),
            pltpu.SemaphoreType.DMA,
            pltpu.SemaphoreType.DMA,
            pltpu.SemaphoreType.DMA,
            pltpu.SemaphoreType.DMA,
            pltpu.SemaphoreType.DMA,
            pltpu.SemaphoreType.DMA,
            pltpu.SemaphoreType.DMA,
        ],
    )
    def sc_kernel(cid_hbm, city_hbm, rev_hbm, ts_hbm, cust_t, city_t, rev_t,
                  cc_t, rt_t, ts_t, revb_hbm, tsb_hbm, consts_hbm, out_hbm,
                  cid_v, city_v, rev_v, ts_v, revb_v, tsb_v, consts_v,
                  revbk_v, cc_i, tsbk_v, rt_i,
                  comb, cust_rows, city_rows, rev_rows, cc_rows, ts_rows, rt_rows,
                  sem_cu, sem_ci, sem_re, sem_cc, sem_ts, sem_rt, sem_o):
        wid = lax.axis_index("s") * NC + lax.axis_index("c")

        pltpu.sync_copy(revb_hbm, revb_v)
        pltpu.sync_copy(tsb_hbm, tsb_v)
        pltpu.sync_copy(consts_hbm, consts_v)
        rev_mean = consts_v[pl.ds(0, 16)]
        rev_std = consts_v[pl.ds(16, 16)]
        ts_mean = consts_v[pl.ds(32, 16)]
        ts_std = consts_v[pl.ds(48, 16)]

        out_h = None
        for p in range(NP):
            base = wid * R + p * RH
            pltpu.sync_copy(cid_hbm.at[pl.ds(base, RH)], cid_v)
            pltpu.sync_copy(city_hbm.at[pl.ds(base, RH)], city_v)
            pltpu.sync_copy(rev_hbm.at[pl.ds(base, RH)], rev_v)
            pltpu.sync_copy(ts_hbm.at[pl.ds(base, RH)], ts_v)

            if out_h is not None:
                out_h.wait()
            g_cust = pltpu.async_copy(cust_t.at[cid_v], cust_rows, sem_cu)
            g_city = pltpu.async_copy(city_t.at[city_v], city_rows, sem_ci)

            def chunk(i, _):
                o = i * 16
                rows = o + lax.iota(jnp.int32, 16)
                cid = cid_v[pl.ds(o, 16)]
                city = city_v[pl.ds(o, 16)]
                rev = rev_v[pl.ds(o, 16)]
                ts = ts_v[pl.ds(o, 16)]

                rbk = _bsearch(revb_v, rev)
                revbk_v[pl.ds(o, 16)] = rbk
                plsc.store_scatter(comb, [rows, jnp.full((16,), C_NORM, jnp.int32)],
                                   (rev - rev_mean) / rev_std)

                cc = ((cid.astype(jnp.uint32) * jnp.uint32(2654435761)) ^
                      (city.astype(jnp.uint32) * jnp.uint32(40503))) % jnp.uint32(5000)
                cc_i[pl.ds(o, 16)] = cc.astype(jnp.int32)

                ts_f = ts.astype(jnp.float32)
                tbk = _bsearch(tsb_v, (ts_f - ts_mean) / ts_std)
                tsbk_v[pl.ds(o, 16)] = tbk
                ts_hour = ts.astype(jnp.uint32) // jnp.uint32(3600)
                rt = ((rbk.astype(jnp.uint32) * jnp.uint32(2654435761)) ^
                      (ts_hour * jnp.uint32(40503))) % jnp.uint32(3000)
                rt_i[pl.ds(o, 16)] = rt.astype(jnp.int32)

                days = ts_f / 86400.0
                dow = jnp.mod(days + 3.0, 7.0) / 7.0
                woy = jnp.mod(days / 7.0, 52.0) / 52.0
                dom = jnp.mod(days, 30.44) / 30.44
                moy = jnp.mod(days / 30.44, 12.0) / 12.0
                col = C_CYC
                for fr in (dow, woy, dom, moy):
                    s, c = _sincos_2pi(fr)
                    plsc.store_scatter(comb, [rows, jnp.full((16,), col, jnp.int32)], s)
                    plsc.store_scatter(comb, [rows, jnp.full((16,), col + 1, jnp.int32)], c)
                    col += 2
                return _

            lax.fori_loop(0, RH // 16, chunk, None)

            g_rev = pltpu.async_copy(rev_t.at[revbk_v], rev_rows, sem_re)
            g_cc = pltpu.async_copy(cc_t.at[cc_i], cc_rows, sem_cc)
            g_ts = pltpu.async_copy(ts_t.at[tsbk_v], ts_rows, sem_ts)
            g_rt = pltpu.async_copy(rt_t.at[rt_i], rt_rows, sem_rt)
            for g in (g_cust, g_city, g_rev, g_cc, g_ts, g_rt):
                g.wait()

            def fixup(i, _):
                o = i * 16
                rows = o + lax.iota(jnp.int32, 16)
                for src, w, c0 in ((cust_rows, D_CUST, 0),
                                   (city_rows, D_CITY, C_CITY),
                                   (rev_rows, D_REV, C_REV),
                                   (cc_rows, D_CC, C_CC), (ts_rows, D_TS, C_TS),
                                   (rt_rows, D_RT, C_RT)):
                    for c in range(w):
                        v = plsc.load_gather(src, [rows, jnp.full((16,), c, jnp.int32)])
                        plsc.store_scatter(
                            comb, [rows, jnp.full((16,), c0 + c, jnp.int32)], v)
                return _

            lax.fori_loop(0, RH // 16, fixup, None)

            out_h = pltpu.async_copy(comb, out_hbm.at[pl.ds(base, RH)], sem_o)
        out_h.wait()

    return sc_kernel


def _pad128(t):
    return jnp.pad(t, ((0, 0), (0, D_PAD - t.shape[1])))


def kernel(customer_id, city_id, revenue, timestamp, cust_table, city_table,
           rev_table, cc_table, rt_table, ts_table, rev_boundaries,
           ts_boundaries, rev_mean, rev_std, ts_mean, ts_std):
    consts = jnp.concatenate([
        jnp.full((16,), rev_mean, jnp.float32),
        jnp.full((16,), rev_std, jnp.float32),
        jnp.full((16,), ts_mean, jnp.float32),
        jnp.full((16,), ts_std, jnp.float32),
    ])
    sc = _make_sc_kernel()
    return sc(customer_id.astype(jnp.int32), city_id.astype(jnp.int32),
              revenue, timestamp.astype(jnp.int32), _pad128(cust_table),
              _pad128(city_table), _pad128(rev_table), _pad128(cc_table),
              _pad128(rt_table), _pad128(ts_table), rev_boundaries,
              ts_boundaries, consts)


# speedup vs baseline: 1.4774x; 1.0018x over previous
import functools

import numpy as np
import jax
import jax.numpy as jnp
from jax import lax
from jax.experimental import pallas as pl
from jax.experimental.pallas import tpu as pltpu, tpu_sc as plsc

B = 16384
D_CUST, D_CITY, D_REV, D_CC, D_TS, D_CYC, D_RT = 64, 16, 32, 16, 32, 8, 12
D_PAD = 128
D_OUT = D_CUST + D_CITY + D_REV + 1 + D_CC + D_TS + D_CYC + D_RT
C_CITY = D_CUST
C_REV = C_CITY + D_CITY
C_NORM = C_REV + D_REV
C_CC = C_NORM + 1
C_TS = C_CC + D_CC
C_CYC = C_TS + D_TS
C_RT = C_CYC + D_CYC
NBOUND = 100

_PI = float(np.pi)


def _bsearch(bounds_ref, v):
    lo = jnp.zeros((16,), jnp.int32)
    w = 64
    while w >= 1:
        cand = lo + w
        g = plsc.load_gather(bounds_ref, [jnp.minimum(cand - 1, NBOUND - 1)])
        ok = (cand <= NBOUND) & (g < v)
        lo = jnp.where(ok, cand, lo)
        w //= 2
    return lo


def _sincos_2pi(f):
    q = f * 4.0
    k = (q + 0.5).astype(jnp.int32)
    t = q - k.astype(jnp.float32)
    th = t * jnp.float32(_PI / 2)
    th2 = th * th
    s = th * (1.0 + th2 * (-1.0 / 6.0 + th2 * (1.0 / 120.0 + th2 * (-1.0 / 5040.0))))
    c = 1.0 + th2 * (-0.5 + th2 * (1.0 / 24.0 + th2 * (-1.0 / 720.0)))
    k2 = k & 3
    sin_o = jnp.where(k2 == 0, s, jnp.where(k2 == 1, c, jnp.where(k2 == 2, -s, -c)))
    cos_o = jnp.where(k2 == 0, c, jnp.where(k2 == 1, -s, jnp.where(k2 == 2, -c, s)))
    return sin_o, cos_o


def _make_sc_kernel():
    info = plsc.get_sparse_core_info()
    NC, NS = info.num_cores, info.num_subcores
    NW = NC * NS
    R = B // NW
    RH = 64
    NP = R // RH
    mesh = plsc.VectorSubcoreMesh(core_axis_name="c", subcore_axis_name="s")

    @functools.partial(
        pl.kernel,
        mesh=mesh,
        out_type=jax.ShapeDtypeStruct((B, D_OUT), jnp.float32),
        compiler_params=pltpu.CompilerParams(use_tc_tiling_on_sc=True,
                                             needs_layout_passes=False),
        scratch_types=[
            pltpu.VMEM((R,), jnp.int32),
            pltpu.VMEM((R,), jnp.int32),
            pltpu.VMEM((R,), jnp.float32),
            pltpu.VMEM((R,), jnp.int32),
            pltpu.VMEM((NBOUND,), jnp.float32),
            pltpu.VMEM((NBOUND,), jnp.float32),
            pltpu.VMEM((64,), jnp.float32),
            pltpu.VMEM((R,), jnp.int32),
            pltpu.VMEM((R,), jnp.int32),
            pltpu.VMEM((R,), jnp.int32),
            pltpu.VMEM((R,), jnp.int32),
            pltpu.VMEM((R,), jnp.float32),
            pltpu.VMEM((R * D_CYC,), jnp.float32),
            pltpu.VMEM((RH, D_OUT), jnp.float32),
            pltpu.VMEM((RH, D_PAD), jnp.float32),
            pltpu.VMEM((RH, D_PAD), jnp.float32),
            pltpu.VMEM((RH, D_PAD), jnp.float32),
            pltpu.VMEM((RH, D_PAD), jnp.float32),
            pltpu.VMEM((RH, D_PAD), jnp.float32),
            pltpu.VMEM((RH, D_PAD), jnp.float32),
            pltpu.SemaphoreType.DMA,
            pltpu.SemaphoreType.DMA,
            pltpu.SemaphoreType.DMA,
            pltpu.SemaphoreType.DMA,
            pltpu.SemaphoreType.DMA,
            pltpu.SemaphoreType.DMA,
            pltpu.SemaphoreType.DMA,
        ],
    )
    def sc_kernel(cid_hbm, city_hbm, rev_hbm, ts_hbm, cust_t, city_t, rev_t,
                  cc_t, rt_t, ts_t, revb_hbm, tsb_hbm, consts_hbm, out_hbm,
                  cid_v, city_v, rev_v, ts_v, revb_v, tsb_v, consts_v,
                  revbk_v, cc_i, tsbk_v, rt_i, norm_v, cyc_v,
                  comb, cust_rows, city_rows, rev_rows, cc_rows, ts_rows, rt_rows,
                  sem_cu, sem_ci, sem_re, sem_cc, sem_ts, sem_rt, sem_o):
        wid = lax.axis_index("s") * NC + lax.axis_index("c")

        pltpu.sync_copy(revb_hbm, revb_v)
        pltpu.sync_copy(tsb_hbm, tsb_v)
        pltpu.sync_copy(consts_hbm, consts_v)
        rev_mean = consts_v[pl.ds(0, 16)]
        rev_std = consts_v[pl.ds(16, 16)]
        ts_mean = consts_v[pl.ds(32, 16)]
        ts_std = consts_v[pl.ds(48, 16)]

        base0 = wid * R
        pltpu.sync_copy(cid_hbm.at[pl.ds(base0, R)], cid_v)
        pltpu.sync_copy(city_hbm.at[pl.ds(base0, R)], city_v)
        pltpu.sync_copy(rev_hbm.at[pl.ds(base0, R)], rev_v)
        pltpu.sync_copy(ts_hbm.at[pl.ds(base0, R)], ts_v)

        def chunk(i, _):
            o = i * 16
            rows = o + lax.iota(jnp.int32, 16)
            cid = cid_v[pl.ds(o, 16)]
            city = city_v[pl.ds(o, 16)]
            rev = rev_v[pl.ds(o, 16)]
            ts = ts_v[pl.ds(o, 16)]

            rbk = _bsearch(revb_v, rev)
            revbk_v[pl.ds(o, 16)] = rbk
            norm_v[pl.ds(o, 16)] = (rev - rev_mean) / rev_std

            cc = ((cid.astype(jnp.uint32) * jnp.uint32(2654435761)) ^
                  (city.astype(jnp.uint32) * jnp.uint32(40503))) % jnp.uint32(5000)
            cc_i[pl.ds(o, 16)] = cc.astype(jnp.int32)

            ts_f = ts.astype(jnp.float32)
            tbk = _bsearch(tsb_v, (ts_f - ts_mean) / ts_std)
            tsbk_v[pl.ds(o, 16)] = tbk
            ts_hour = ts.astype(jnp.uint32) // jnp.uint32(3600)
            rt = ((rbk.astype(jnp.uint32) * jnp.uint32(2654435761)) ^
                  (ts_hour * jnp.uint32(40503))) % jnp.uint32(3000)
            rt_i[pl.ds(o, 16)] = rt.astype(jnp.int32)

            days = ts_f / 86400.0
            dow = jnp.mod(days + 3.0, 7.0) / 7.0
            woy = jnp.mod(days / 7.0, 52.0) / 52.0
            dom = jnp.mod(days, 30.44) / 30.44
            moy = jnp.mod(days / 30.44, 12.0) / 12.0
            flat = rows * D_CYC
            col = 0
            for fr in (dow, woy, dom, moy):
                s, c = _sincos_2pi(fr)
                plsc.store_scatter(cyc_v, [flat + col], s)
                plsc.store_scatter(cyc_v, [flat + col + 1], c)
                col += 2
            return _

        lax.fori_loop(0, R // 16, chunk, None)

        out_h = None
        for p in range(NP):
            base = base0 + p * RH
            sl = pl.ds(p * RH, RH)
            g_cust = pltpu.async_copy(cust_t.at[cid_v.at[sl]], cust_rows, sem_cu)
            g_city = pltpu.async_copy(city_t.at[city_v.at[sl]], city_rows, sem_ci)
            g_rev = pltpu.async_copy(rev_t.at[revbk_v.at[sl]], rev_rows, sem_re)
            g_cc = pltpu.async_copy(cc_t.at[cc_i.at[sl]], cc_rows, sem_cc)
            g_ts = pltpu.async_copy(ts_t.at[tsbk_v.at[sl]], ts_rows, sem_ts)
            g_rt = pltpu.async_copy(rt_t.at[rt_i.at[sl]], rt_rows, sem_rt)
            if out_h is not None:
                out_h.wait()
            for g in (g_cust, g_city, g_rev, g_cc, g_ts, g_rt):
                g.wait()

            def fixup(i, _):
                o = i * 16
                rows = o + lax.iota(jnp.int32, 16)
                grows = p * RH + o + lax.iota(jnp.int32, 16)
                for src, w, c0 in ((cust_rows, D_CUST, 0),
                                   (city_rows, D_CITY, C_CITY),
                                   (rev_rows, D_REV, C_REV),
                                   (cc_rows, D_CC, C_CC), (ts_rows, D_TS, C_TS),
                                   (rt_rows, D_RT, C_RT)):
                    for c in range(w):
                        v = plsc.load_gather(src, [rows, jnp.full((16,), c, jnp.int32)])
                        plsc.store_scatter(
                            comb, [rows, jnp.full((16,), c0 + c, jnp.int32)], v)
                nv = norm_v[pl.ds(p * RH + o, 16)]
                plsc.store_scatter(comb, [rows, jnp.full((16,), C_NORM, jnp.int32)], nv)
                for c in range(D_CYC):
                    v = plsc.load_gather(cyc_v, [grows * D_CYC + c])
                    plsc.store_scatter(
                        comb, [rows, jnp.full((16,), C_CYC + c, jnp.int32)], v)
                return _

            lax.fori_loop(0, RH // 16, fixup, None)

            out_h = pltpu.async_copy(comb, out_hbm.at[pl.ds(base, RH)], sem_o)
        out_h.wait()

    return sc_kernel


def _pad128(t):
    return jnp.pad(t, ((0, 0), (0, D_PAD - t.shape[1])))


def kernel(customer_id, city_id, revenue, timestamp, cust_table, city_table,
           rev_table, cc_table, rt_table, ts_table, rev_boundaries,
           ts_boundaries, rev_mean, rev_std, ts_mean, ts_std):
    consts = jnp.concatenate([
        jnp.full((16,), rev_mean, jnp.float32),
        jnp.full((16,), rev_std, jnp.float32),
        jnp.full((16,), ts_mean, jnp.float32),
        jnp.full((16,), ts_std, jnp.float32),
    ])
    sc = _make_sc_kernel()
    return sc(customer_id.astype(jnp.int32), city_id.astype(jnp.int32),
              revenue, timestamp.astype(jnp.int32), _pad128(cust_table),
              _pad128(city_table), _pad128(rev_table), _pad128(cc_table),
              _pad128(rt_table), _pad128(ts_table), rev_boundaries,
              ts_boundaries, consts)


# speedup vs baseline: 1.5101x; 1.0221x over previous
import functools

import numpy as np
import jax
import jax.numpy as jnp
from jax import lax
from jax.experimental import pallas as pl
from jax.experimental.pallas import tpu as pltpu, tpu_sc as plsc

B = 16384
D_CUST, D_CITY, D_REV, D_CC, D_TS, D_CYC, D_RT = 64, 16, 32, 16, 32, 8, 12
D_PAD = 128
D_OUT = D_CUST + D_CITY + D_REV + 1 + D_CC + D_TS + D_CYC + D_RT
C_CITY = D_CUST
C_REV = C_CITY + D_CITY
C_NORM = C_REV + D_REV
C_CC = C_NORM + 1
C_TS = C_CC + D_CC
C_CYC = C_TS + D_TS
C_RT = C_CYC + D_CYC
NBOUND = 100

_PI = float(np.pi)


def _bsearch(bounds_ref, v):
    lo = jnp.zeros((16,), jnp.int32)
    w = 64
    while w >= 1:
        cand = lo + w
        g = plsc.load_gather(bounds_ref, [jnp.minimum(cand - 1, NBOUND - 1)])
        ok = (cand <= NBOUND) & (g < v)
        lo = jnp.where(ok, cand, lo)
        w //= 2
    return lo


def _sincos_2pi(f):
    q = f * 4.0
    k = (q + 0.5).astype(jnp.int32)
    t = q - k.astype(jnp.float32)
    th = t * jnp.float32(_PI / 2)
    th2 = th * th
    s = th * (1.0 + th2 * (-1.0 / 6.0 + th2 * (1.0 / 120.0 + th2 * (-1.0 / 5040.0))))
    c = 1.0 + th2 * (-0.5 + th2 * (1.0 / 24.0 + th2 * (-1.0 / 720.0)))
    k2 = k & 3
    sin_o = jnp.where(k2 == 0, s, jnp.where(k2 == 1, c, jnp.where(k2 == 2, -s, -c)))
    cos_o = jnp.where(k2 == 0, c, jnp.where(k2 == 1, -s, jnp.where(k2 == 2, -c, s)))
    return sin_o, cos_o


def _make_sc_kernel():
    info = plsc.get_sparse_core_info()
    NC, NS = info.num_cores, info.num_subcores
    NW = NC * NS
    R = B // NW
    RH = 64
    NP = R // RH
    mesh = plsc.VectorSubcoreMesh(core_axis_name="c", subcore_axis_name="s")

    @functools.partial(
        pl.kernel,
        mesh=mesh,
        out_type=jax.ShapeDtypeStruct((B, D_OUT), jnp.float32),
        compiler_params=pltpu.CompilerParams(use_tc_tiling_on_sc=True,
                                             needs_layout_passes=False),
        scratch_types=[
            pltpu.VMEM((R,), jnp.int32),
            pltpu.VMEM((R,), jnp.int32),
            pltpu.VMEM((R,), jnp.float32),
            pltpu.VMEM((R,), jnp.int32),
            pltpu.VMEM((NBOUND,), jnp.float32),
            pltpu.VMEM((NBOUND,), jnp.float32),
            pltpu.VMEM((64,), jnp.float32),
            pltpu.VMEM((R,), jnp.int32),
            pltpu.VMEM((R,), jnp.int32),
            pltpu.VMEM((R,), jnp.int32),
            pltpu.VMEM((R,), jnp.int32),
            pltpu.VMEM((R,), jnp.float32),
            pltpu.VMEM((R * D_CYC,), jnp.float32),
            pltpu.VMEM((RH, D_OUT), jnp.float32),
            *([pltpu.VMEM((RH, D_PAD), jnp.float32)] * 12),
            *([pltpu.SemaphoreType.DMA] * 13),
        ],
    )
    def sc_kernel(cid_hbm, city_hbm, rev_hbm, ts_hbm, cust_t, city_t, rev_t,
                  cc_t, rt_t, ts_t, revb_hbm, tsb_hbm, consts_hbm, out_hbm,
                  cid_v, city_v, rev_v, ts_v, revb_v, tsb_v, consts_v,
                  revbk_v, cc_i, tsbk_v, rt_i, norm_v, cyc_v, comb, *bufs_sems):
        dsts = [bufs_sems[0:6], bufs_sems[6:12]]
        sems = [bufs_sems[12:18], bufs_sems[18:24]]
        sem_o = bufs_sems[24]
        wid = lax.axis_index("s") * NC + lax.axis_index("c")

        pltpu.sync_copy(revb_hbm, revb_v)
        pltpu.sync_copy(tsb_hbm, tsb_v)
        pltpu.sync_copy(consts_hbm, consts_v)
        rev_mean = consts_v[pl.ds(0, 16)]
        rev_std = consts_v[pl.ds(16, 16)]
        ts_mean = consts_v[pl.ds(32, 16)]
        ts_std = consts_v[pl.ds(48, 16)]

        base0 = wid * R
        pltpu.sync_copy(cid_hbm.at[pl.ds(base0, R)], cid_v)
        pltpu.sync_copy(city_hbm.at[pl.ds(base0, R)], city_v)
        pltpu.sync_copy(rev_hbm.at[pl.ds(base0, R)], rev_v)
        pltpu.sync_copy(ts_hbm.at[pl.ds(base0, R)], ts_v)

        def chunk(i, _):
            o = i * 16
            rows = o + lax.iota(jnp.int32, 16)
            cid = cid_v[pl.ds(o, 16)]
            city = city_v[pl.ds(o, 16)]
            rev = rev_v[pl.ds(o, 16)]
            ts = ts_v[pl.ds(o, 16)]

            rbk = _bsearch(revb_v, rev)
            revbk_v[pl.ds(o, 16)] = rbk
            norm_v[pl.ds(o, 16)] = (rev - rev_mean) / rev_std

            cc = ((cid.astype(jnp.uint32) * jnp.uint32(2654435761)) ^
                  (city.astype(jnp.uint32) * jnp.uint32(40503))) % jnp.uint32(5000)
            cc_i[pl.ds(o, 16)] = cc.astype(jnp.int32)

            ts_f = ts.astype(jnp.float32)
            tbk = _bsearch(tsb_v, (ts_f - ts_mean) / ts_std)
            tsbk_v[pl.ds(o, 16)] = tbk
            ts_hour = ts.astype(jnp.uint32) // jnp.uint32(3600)
            rt = ((rbk.astype(jnp.uint32) * jnp.uint32(2654435761)) ^
                  (ts_hour * jnp.uint32(40503))) % jnp.uint32(3000)
            rt_i[pl.ds(o, 16)] = rt.astype(jnp.int32)

            days = ts_f / 86400.0
            dow = jnp.mod(days + 3.0, 7.0) / 7.0
            woy = jnp.mod(days / 7.0, 52.0) / 52.0
            dom = jnp.mod(days, 30.44) / 30.44
            moy = jnp.mod(days / 30.44, 12.0) / 12.0
            flat = rows * D_CYC
            col = 0
            for fr in (dow, woy, dom, moy):
                s, c = _sincos_2pi(fr)
                plsc.store_scatter(cyc_v, [flat + col], s)
                plsc.store_scatter(cyc_v, [flat + col + 1], c)
                col += 2
            return _

        lax.fori_loop(0, R // 16, chunk, None)

        tables = (cust_t, city_t, rev_t, cc_t, ts_t, rt_t)
        idx_refs = (cid_v, city_v, revbk_v, cc_i, tsbk_v, rt_i)

        def fire(p):
            s = p % 2
            sl = pl.ds(p * RH, RH)
            return [pltpu.async_copy(tab.at[idxr.at[sl]], d, sem)
                    for tab, idxr, d, sem
                    in zip(tables, idx_refs, dsts[s], sems[s])]

        pend = fire(0)
        out_h = None
        for p in range(NP):
            nxt = fire(p + 1) if p + 1 < NP else None
            for g in pend:
                g.wait()
            if out_h is not None:
                out_h.wait()
            s = p % 2
            cust_rows, city_rows, rev_rows, cc_rows, ts_rows, rt_rows = dsts[s]

            def fixup(i, _):
                o = i * 16
                rows = o + lax.iota(jnp.int32, 16)
                grows = p * RH + o + lax.iota(jnp.int32, 16)
                for src, w, c0 in ((cust_rows, D_CUST, 0),
                                   (city_rows, D_CITY, C_CITY),
                                   (rev_rows, D_REV, C_REV),
                                   (cc_rows, D_CC, C_CC), (ts_rows, D_TS, C_TS),
                                   (rt_rows, D_RT, C_RT)):
                    for c in range(w):
                        v = plsc.load_gather(src, [rows, jnp.full((16,), c, jnp.int32)])
                        plsc.store_scatter(
                            comb, [rows, jnp.full((16,), c0 + c, jnp.int32)], v)
                nv = norm_v[pl.ds(p * RH + o, 16)]
                plsc.store_scatter(comb, [rows, jnp.full((16,), C_NORM, jnp.int32)], nv)
                for c in range(D_CYC):
                    v = plsc.load_gather(cyc_v, [grows * D_CYC + c])
                    plsc.store_scatter(
                        comb, [rows, jnp.full((16,), C_CYC + c, jnp.int32)], v)
                return _

            lax.fori_loop(0, RH // 16, fixup, None)

            out_h = pltpu.async_copy(comb, out_hbm.at[pl.ds(base0 + p * RH, RH)],
                                     sem_o)
            pend = nxt
        out_h.wait()

    return sc_kernel


def _pad128(t):
    return jnp.pad(t, ((0, 0), (0, D_PAD - t.shape[1])))


def kernel(customer_id, city_id, revenue, timestamp, cust_table, city_table,
           rev_table, cc_table, rt_table, ts_table, rev_boundaries,
           ts_boundaries, rev_mean, rev_std, ts_mean, ts_std):
    consts = jnp.concatenate([
        jnp.full((16,), rev_mean, jnp.float32),
        jnp.full((16,), rev_std, jnp.float32),
        jnp.full((16,), ts_mean, jnp.float32),
        jnp.full((16,), ts_std, jnp.float32),
    ])
    sc = _make_sc_kernel()
    return sc(customer_id.astype(jnp.int32), city_id.astype(jnp.int32),
              revenue, timestamp.astype(jnp.int32), _pad128(cust_table),
              _pad128(city_table), _pad128(rev_table), _pad128(cc_table),
              _pad128(rt_table), _pad128(ts_table), rev_boundaries,
              ts_boundaries, consts)


# speedup vs baseline: 1.5151x; 1.0033x over previous
import functools

import numpy as np
import jax
import jax.numpy as jnp
from jax import lax
from jax.experimental import pallas as pl
from jax.experimental.pallas import tpu as pltpu, tpu_sc as plsc

B = 16384
D_CUST, D_CITY, D_REV, D_CC, D_TS, D_CYC, D_RT = 64, 16, 32, 16, 32, 8, 12
D_PAD = 128
D_OUT = D_CUST + D_CITY + D_REV + 1 + D_CC + D_TS + D_CYC + D_RT
C_CITY = D_CUST
C_REV = C_CITY + D_CITY
C_NORM = C_REV + D_REV
C_CC = C_NORM + 1
C_TS = C_CC + D_CC
C_CYC = C_TS + D_TS
C_RT = C_CYC + D_CYC
NBOUND = 100

_PI = float(np.pi)


def _bsearch(bounds_ref, v):
    lo = jnp.zeros((16,), jnp.int32)
    w = 64
    while w >= 1:
        cand = lo + w
        g = plsc.load_gather(bounds_ref, [jnp.minimum(cand - 1, NBOUND - 1)])
        ok = (cand <= NBOUND) & (g < v)
        lo = jnp.where(ok, cand, lo)
        w //= 2
    return lo


def _sincos_2pi(f):
    q = f * 4.0
    k = (q + 0.5).astype(jnp.int32)
    t = q - k.astype(jnp.float32)
    th = t * jnp.float32(_PI / 2)
    th2 = th * th
    s = th * (1.0 + th2 * (-1.0 / 6.0 + th2 * (1.0 / 120.0 + th2 * (-1.0 / 5040.0))))
    c = 1.0 + th2 * (-0.5 + th2 * (1.0 / 24.0 + th2 * (-1.0 / 720.0)))
    k2 = k & 3
    sin_o = jnp.where(k2 == 0, s, jnp.where(k2 == 1, c, jnp.where(k2 == 2, -s, -c)))
    cos_o = jnp.where(k2 == 0, c, jnp.where(k2 == 1, -s, jnp.where(k2 == 2, -c, s)))
    return sin_o, cos_o


def _make_sc_kernel():
    info = plsc.get_sparse_core_info()
    NC, NS = info.num_cores, info.num_subcores
    NW = NC * NS
    R = B // NW
    RH = 64
    NP = R // RH
    mesh = plsc.VectorSubcoreMesh(core_axis_name="c", subcore_axis_name="s")

    @functools.partial(
        pl.kernel,
        mesh=mesh,
        out_type=jax.ShapeDtypeStruct((B, D_OUT), jnp.float32),
        compiler_params=pltpu.CompilerParams(use_tc_tiling_on_sc=True,
                                             needs_layout_passes=False),
        scratch_types=[
            pltpu.VMEM((R,), jnp.int32),
            pltpu.VMEM((R,), jnp.int32),
            pltpu.VMEM((R,), jnp.float32),
            pltpu.VMEM((R,), jnp.int32),
            pltpu.VMEM((NBOUND,), jnp.float32),
            pltpu.VMEM((NBOUND,), jnp.float32),
            pltpu.VMEM((64,), jnp.float32),
            pltpu.VMEM((R,), jnp.int32),
            pltpu.VMEM((R,), jnp.int32),
            pltpu.VMEM((R,), jnp.int32),
            pltpu.VMEM((R,), jnp.int32),
            pltpu.VMEM((R,), jnp.float32),
            pltpu.VMEM((R * D_CYC,), jnp.float32),
            pltpu.VMEM((RH, D_OUT), jnp.float32),
            *([pltpu.VMEM((RH, D_PAD), jnp.float32)] * 12),
            *([pltpu.SemaphoreType.DMA] * 13),
        ],
    )
    def sc_kernel(cid_hbm, city_hbm, rev_hbm, ts_hbm, cust_t, city_t, rev_t,
                  cc_t, rt_t, ts_t, revb_hbm, tsb_hbm, consts_hbm, out_hbm,
                  cid_v, city_v, rev_v, ts_v, revb_v, tsb_v, consts_v,
                  revbk_v, cc_i, tsbk_v, rt_i, norm_v, cyc_v, comb, *bufs_sems):
        dsts = [bufs_sems[0:6], bufs_sems[6:12]]
        sems = [bufs_sems[12:18], bufs_sems[18:24]]
        sem_o = bufs_sems[24]
        wid = lax.axis_index("s") * NC + lax.axis_index("c")

        pltpu.sync_copy(revb_hbm, revb_v)
        pltpu.sync_copy(tsb_hbm, tsb_v)
        pltpu.sync_copy(consts_hbm, consts_v)
        rev_mean = consts_v[pl.ds(0, 16)]
        rev_std = consts_v[pl.ds(16, 16)]
        ts_mean = consts_v[pl.ds(32, 16)]
        ts_std = consts_v[pl.ds(48, 16)]

        base0 = wid * R
        pltpu.sync_copy(cid_hbm.at[pl.ds(base0, R)], cid_v)
        pltpu.sync_copy(city_hbm.at[pl.ds(base0, R)], city_v)
        pltpu.sync_copy(rev_hbm.at[pl.ds(base0, R)], rev_v)
        pltpu.sync_copy(ts_hbm.at[pl.ds(base0, R)], ts_v)

        tables = (cust_t, city_t, rev_t, cc_t, ts_t, rt_t)
        idx_refs = (cid_v, city_v, revbk_v, cc_i, tsbk_v, rt_i)

        def fire(p, lo=0, hi=6):
            s = p % 2
            sl = pl.ds(p * RH, RH)
            return [pltpu.async_copy(tab.at[idxr.at[sl]], d, sem)
                    for tab, idxr, d, sem
                    in zip(tables[lo:hi], idx_refs[lo:hi],
                           dsts[s][lo:hi], sems[s][lo:hi])]

        pre0 = fire(0, 0, 2)
        pre1 = fire(1, 0, 2)

        def chunk(i, _):
            o = i * 16
            rows = o + lax.iota(jnp.int32, 16)
            cid = cid_v[pl.ds(o, 16)]
            city = city_v[pl.ds(o, 16)]
            rev = rev_v[pl.ds(o, 16)]
            ts = ts_v[pl.ds(o, 16)]

            rbk = _bsearch(revb_v, rev)
            revbk_v[pl.ds(o, 16)] = rbk
            norm_v[pl.ds(o, 16)] = (rev - rev_mean) / rev_std

            cc = ((cid.astype(jnp.uint32) * jnp.uint32(2654435761)) ^
                  (city.astype(jnp.uint32) * jnp.uint32(40503))) % jnp.uint32(5000)
            cc_i[pl.ds(o, 16)] = cc.astype(jnp.int32)

            ts_f = ts.astype(jnp.float32)
            tbk = _bsearch(tsb_v, (ts_f - ts_mean) / ts_std)
            tsbk_v[pl.ds(o, 16)] = tbk
            ts_hour = ts.astype(jnp.uint32) // jnp.uint32(3600)
            rt = ((rbk.astype(jnp.uint32) * jnp.uint32(2654435761)) ^
                  (ts_hour * jnp.uint32(40503))) % jnp.uint32(3000)
            rt_i[pl.ds(o, 16)] = rt.astype(jnp.int32)

            days = ts_f / 86400.0
            dow = jnp.mod(days + 3.0, 7.0) / 7.0
            woy = jnp.mod(days / 7.0, 52.0) / 52.0
            dom = jnp.mod(days, 30.44) / 30.44
            moy = jnp.mod(days / 30.44, 12.0) / 12.0
            flat = rows * D_CYC
            col = 0
            for fr in (dow, woy, dom, moy):
                s, c = _sincos_2pi(fr)
                plsc.store_scatter(cyc_v, [flat + col], s)
                plsc.store_scatter(cyc_v, [flat + col + 1], c)
                col += 2
            return _

        lax.fori_loop(0, R // 16, chunk, None)

        pend = pre0 + fire(0, 2, 6)
        pre = {1: pre1}
        out_h = None
        for p in range(NP):
            nxt = (pre.pop(p + 1) + fire(p + 1, 2, 6)) if p + 1 < NP else None
            for g in pend:
                g.wait()
            if out_h is not None:
                out_h.wait()
            s = p % 2
            cust_rows, city_rows, rev_rows, cc_rows, ts_rows, rt_rows = dsts[s]

            def fixup(i, _):
                o = i * 16
                rows = o + lax.iota(jnp.int32, 16)
                grows = p * RH + o + lax.iota(jnp.int32, 16)
                for src, w, c0 in ((cust_rows, D_CUST, 0),
                                   (city_rows, D_CITY, C_CITY),
                                   (rev_rows, D_REV, C_REV),
                                   (cc_rows, D_CC, C_CC), (ts_rows, D_TS, C_TS),
                                   (rt_rows, D_RT, C_RT)):
                    for c in range(w):
                        v = plsc.load_gather(src, [rows, jnp.full((16,), c, jnp.int32)])
                        plsc.store_scatter(
                            comb, [rows, jnp.full((16,), c0 + c, jnp.int32)], v)
                nv = norm_v[pl.ds(p * RH + o, 16)]
                plsc.store_scatter(comb, [rows, jnp.full((16,), C_NORM, jnp.int32)], nv)
                for c in range(D_CYC):
                    v = plsc.load_gather(cyc_v, [grows * D_CYC + c])
                    plsc.store_scatter(
                        comb, [rows, jnp.full((16,), C_CYC + c, jnp.int32)], v)
                return _

            lax.fori_loop(0, RH // 16, fixup, None)

            if p + 2 < NP:
                pre[p + 2] = fire(p + 2, 0, 2)
            out_h = pltpu.async_copy(comb, out_hbm.at[pl.ds(base0 + p * RH, RH)],
                                     sem_o)
            pend = nxt
        out_h.wait()

    return sc_kernel


def _pad128(t):
    return jnp.pad(t, ((0, 0), (0, D_PAD - t.shape[1])))


def kernel(customer_id, city_id, revenue, timestamp, cust_table, city_table,
           rev_table, cc_table, rt_table, ts_table, rev_boundaries,
           ts_boundaries, rev_mean, rev_std, ts_mean, ts_std):
    consts = jnp.concatenate([
        jnp.full((16,), rev_mean, jnp.float32),
        jnp.full((16,), rev_std, jnp.float32),
        jnp.full((16,), ts_mean, jnp.float32),
        jnp.full((16,), ts_std, jnp.float32),
    ])
    sc = _make_sc_kernel()
    return sc(customer_id.astype(jnp.int32), city_id.astype(jnp.int32),
              revenue, timestamp.astype(jnp.int32), _pad128(cust_table),
              _pad128(city_table), _pad128(rev_table), _pad128(cc_table),
              _pad128(rt_table), _pad128(ts_table), rev_boundaries,
              ts_boundaries, consts)
